# baseline (speedup 1.0000x reference)
import functools

import jax
import jax.numpy as jnp
from jax import lax
from jax.experimental import pallas as pl
from jax.experimental.pallas import tpu as pltpu
from jax.experimental.pallas import tpu_sc as plsc

_INFO = plsc.get_sparse_core_info()
_NC = _INFO.num_cores
_NS = _INFO.num_subcores
_NW = _NC * _NS
_CH = 128


@functools.lru_cache(maxsize=None)
def _build_gather(V, D, n_chunks):
  mesh = plsc.VectorSubcoreMesh(core_axis_name="c", subcore_axis_name="s")

  @functools.partial(
      pl.kernel,
      mesh=mesh,
      out_type=jax.ShapeDtypeStruct((_NW, n_chunks, _CH, D), jnp.float32),
      scratch_types=[
          pltpu.VMEM((n_chunks, _CH), jnp.int32),
          pltpu.VMEM((_CH, D), jnp.float32),
          pltpu.SemaphoreType.DMA,
      ],
      compiler_params=pltpu.CompilerParams(use_tc_tiling_on_sc=False),
  )
  def gather_kernel(idx_hbm, table_hbm, out_hbm, idx_v, rows_v, sem):
    wid = lax.axis_index("s") * _NC + lax.axis_index("c")
    pltpu.sync_copy(idx_hbm.at[wid], idx_v)

    def step(j, carry):
      pltpu.async_copy(table_hbm.at[idx_v.at[j]], rows_v, sem).wait()
      pltpu.sync_copy(rows_v, out_hbm.at[wid, j])
      return carry

    lax.fori_loop(0, n_chunks, step, 0)

  return gather_kernel


def kernel(x, table):
  B, S = x.shape
  V, D = table.shape
  n = B * S
  flat = x.reshape(n).astype(jnp.int32)

  block = _NW * _CH
  n_pad = -(-n // block) * block
  if n_pad != n:
    flat = jnp.pad(flat, (0, n_pad - n))
  n_chunks = n_pad // block

  idx = flat.reshape(_NW, n_chunks, _CH)
  out = _build_gather(V, D, n_chunks)(idx, table)
  out = out.reshape(n_pad, D)
  if n_pad != n:
    out = out[:n]
  return out.reshape(B, S, D)


# speedup vs baseline: 1.1115x; 1.1115x over previous
import functools

import jax
import jax.numpy as jnp
from jax import lax
from jax.experimental import pallas as pl
from jax.experimental.pallas import tpu as pltpu
from jax.experimental.pallas import tpu_sc as plsc

_INFO = plsc.get_sparse_core_info()
_NC = _INFO.num_cores
_NS = _INFO.num_subcores
_NW = _NC * _NS
_CH = 128


_NBUF = 8


@functools.lru_cache(maxsize=None)
def _build_gather(V, D, n_chunks):
  assert n_chunks % _NBUF == 0
  laps = n_chunks // _NBUF
  mesh = plsc.VectorSubcoreMesh(core_axis_name="c", subcore_axis_name="s")

  @functools.partial(
      pl.kernel,
      mesh=mesh,
      out_type=jax.ShapeDtypeStruct((_NW, n_chunks, _CH, D), jnp.float32),
      scratch_types=[
          pltpu.VMEM((n_chunks, _CH), jnp.int32),
          pltpu.VMEM((_NBUF, _CH, D), jnp.float32),
          pltpu.SemaphoreType.DMA((_NBUF,)),
          pltpu.SemaphoreType.DMA((_NBUF,)),
      ],
      compiler_params=pltpu.CompilerParams(use_tc_tiling_on_sc=False),
  )
  def gather_kernel(idx_hbm, table_hbm, out_hbm, idx_v, rows_v, gsem, wsem):
    wid = lax.axis_index("s") * _NC + lax.axis_index("c")
    pltpu.sync_copy(idx_hbm.at[wid], idx_v)

    def start_gather(j, b):
      pltpu.make_async_copy(
          table_hbm.at[idx_v.at[j]], rows_v.at[b], gsem.at[b]).start()

    def wait_gather(b):
      pltpu.make_async_copy(
          table_hbm.at[idx_v.at[0]], rows_v.at[b], gsem.at[b]).wait()

    def start_write(j, b):
      pltpu.make_async_copy(
          rows_v.at[b], out_hbm.at[wid, j], wsem.at[b]).start()

    def wait_write(b):
      pltpu.make_async_copy(
          rows_v.at[b], out_hbm.at[wid, 0], wsem.at[b]).wait()

    for b in range(_NBUF):
      start_gather(b, b)

    def lap(g, carry):
      j0 = g * _NBUF
      for b in range(_NBUF):
        wait_gather(b)
        start_write(j0 + b, b)
      for b in range(_NBUF):
        wait_write(b)
        start_gather(j0 + _NBUF + b, b)
      return carry

    lax.fori_loop(0, laps - 1, lap, 0)

    j0 = (laps - 1) * _NBUF
    for b in range(_NBUF):
      wait_gather(b)
      start_write(j0 + b, b)
    for b in range(_NBUF):
      wait_write(b)

  return gather_kernel


def kernel(x, table):
  B, S = x.shape
  V, D = table.shape
  n = B * S
  flat = x.reshape(n).astype(jnp.int32)

  block = _NW * _CH * _NBUF
  n_pad = -(-n // block) * block
  if n_pad != n:
    flat = jnp.pad(flat, (0, n_pad - n))
  n_chunks = n_pad // (_NW * _CH)

  idx = flat.reshape(_NW, n_chunks, _CH)
  out = _build_gather(V, D, n_chunks)(idx, table)
  out = out.reshape(n_pad, D)
  if n_pad != n:
    out = out[:n]
  return out.reshape(B, S, D)


# speedup vs baseline: 1.1133x; 1.0017x over previous
import functools

import jax
import jax.numpy as jnp
from jax import lax
from jax.experimental import pallas as pl
from jax.experimental.pallas import tpu as pltpu
from jax.experimental.pallas import tpu_sc as plsc

_INFO = plsc.get_sparse_core_info()
_NC = _INFO.num_cores
_NS = _INFO.num_subcores
_NW = _NC * _NS
_CH = 128


_NBUF = 8


@functools.lru_cache(maxsize=None)
def _build_gather(V, D, n_chunks):
  assert n_chunks % _NBUF == 0 and n_chunks >= 2 * _NBUF
  laps = n_chunks // _NBUF
  mesh = plsc.VectorSubcoreMesh(core_axis_name="c", subcore_axis_name="s")

  @functools.partial(
      pl.kernel,
      mesh=mesh,
      out_type=jax.ShapeDtypeStruct((_NW, n_chunks, _CH, D), jnp.float32),
      scratch_types=[
          pltpu.VMEM((n_chunks, _CH), jnp.int32),
          pltpu.VMEM((_NBUF, _CH, D), jnp.float32),
          pltpu.SemaphoreType.DMA((_NBUF,)),
          pltpu.SemaphoreType.DMA((_NBUF,)),
      ],
      compiler_params=pltpu.CompilerParams(use_tc_tiling_on_sc=False),
  )
  def gather_kernel(idx_hbm, table_hbm, out_hbm, idx_v, rows_v, gsem, wsem):
    wid = lax.axis_index("s") * _NC + lax.axis_index("c")
    pltpu.sync_copy(idx_hbm.at[wid], idx_v)

    def start_gather(j, b):
      pltpu.make_async_copy(
          table_hbm.at[idx_v.at[j]], rows_v.at[b], gsem.at[b]).start()

    def wait_gather(b):
      pltpu.make_async_copy(
          table_hbm.at[idx_v.at[0]], rows_v.at[b], gsem.at[b]).wait()

    def start_write(j, b):
      pltpu.make_async_copy(
          rows_v.at[b], out_hbm.at[wid, j], wsem.at[b]).start()

    def wait_write(b):
      pltpu.make_async_copy(
          rows_v.at[b], out_hbm.at[wid, 0], wsem.at[b]).wait()

    H = _NBUF // 2

    for b in range(H):
      start_gather(b, b)

    for b in range(_NBUF):
      bn = (b + H) % _NBUF
      wait_gather(b)
      start_write(b, b)
      if b >= H:
        wait_write(bn)
      start_gather(b + H, bn)

    def lap(g, carry):
      j0 = g * _NBUF
      for b in range(_NBUF):
        bn = (b + H) % _NBUF
        wait_gather(b)
        start_write(j0 + b, b)
        wait_write(bn)
        start_gather(j0 + b + H, bn)
      return carry

    lax.fori_loop(1, laps - 1, lap, 0)

    j0 = (laps - 1) * _NBUF
    for b in range(_NBUF):
      bn = (b + H) % _NBUF
      wait_gather(b)
      start_write(j0 + b, b)
      wait_write(bn)
      if b < H:
        start_gather(j0 + b + H, bn)
    for b in range(H, _NBUF):
      wait_write(b)

  return gather_kernel


def kernel(x, table):
  B, S = x.shape
  V, D = table.shape
  n = B * S
  flat = x.reshape(n).astype(jnp.int32)

  block = _NW * _CH * _NBUF
  n_pad = -(-n // block) * block
  if n_pad != n:
    flat = jnp.pad(flat, (0, n_pad - n))
  n_chunks = n_pad // (_NW * _CH)

  idx = flat.reshape(_NW, n_chunks, _CH)
  out = _build_gather(V, D, n_chunks)(idx, table)
  out = out.reshape(n_pad, D)
  if n_pad != n:
    out = out[:n]
  return out.reshape(B, S, D)
